# baseline (speedup 1.0000x reference)
import functools
import jax
import jax.numpy as jnp
from jax import lax
from jax.experimental import pallas as pl
from jax.experimental.pallas import tpu as pltpu
from jax.experimental.pallas import tpu_sc as plsc

B = 16
N = 32768
H = 128
BLK = 1024
NBLK = N // BLK
NCORES = 2
NTILES = 32
TOK = N // NTILES
SUB = 256
NSUB = TOK // SUB
NV = H // 16

_NEG = -1e30


def _sc_gather_body(x_hbm, last_hbm, vi_hbm, idx_v, rows_v, sem):
    cid = lax.axis_index("c")
    sid = lax.axis_index("s")

    @pl.when(jnp.logical_and(cid == 0, sid == 0))
    def _():
        pltpu.sync_copy(last_hbm, idx_v)
        pltpu.async_copy(x_hbm.at[idx_v], rows_v, sem).wait()
        pltpu.sync_copy(rows_v, vi_hbm)


def _tc_body(x_ref, seg_ref, vi_ref, W1_ref, b12_ref, W2_ref, qw_ref, qb_ref,
             lg_ref, m_ref, s_ref, hist_ref, u_sc, m_sc, s_sc):
    i = pl.program_id(0)

    @pl.when(i == 0)
    def _init():
        u_sc[...] = jnp.dot(vi_ref[...], W1_ref[...],
                            preferred_element_type=jnp.float32) + b12_ref[...]
        m_sc[...] = jnp.full((1, B), _NEG, jnp.float32)
        s_sc[...] = jnp.zeros((1, B), jnp.float32)

    x = x_ref[...]
    seg = seg_ref[...]
    iota = lax.broadcasted_iota(jnp.int32, (BLK, B), 1)
    ohb = seg == iota
    oh = ohb.astype(jnp.float32)

    z = jnp.dot(x, W2_ref[...], preferred_element_type=jnp.float32)
    z = z + jnp.dot(oh, u_sc[...], preferred_element_type=jnp.float32)
    h = jax.nn.sigmoid(z)
    lg = jnp.dot(h, qw_ref[...], preferred_element_type=jnp.float32) + qb_ref[...]
    lg_ref[...] = lg

    hist_ref[...] = jnp.sum(oh, axis=0, keepdims=True)[None]

    m_old = m_sc[...]
    mb = jnp.max(jnp.where(ohb, lg, _NEG), axis=0, keepdims=True)
    m_new = jnp.maximum(m_old, mb)
    m_sc[...] = m_new

    mtok = jnp.sum(oh * m_new, axis=1, keepdims=True)
    ex = jnp.exp(lg - mtok)
    sum_b = jnp.sum(oh * ex, axis=0, keepdims=True)
    scale = jnp.exp(m_old - m_new)
    s_sc[...] = s_sc[...] * scale + sum_b

    @pl.when(i == NBLK - 1)
    def _fin():
        m_ref[...] = m_sc[...]
        s_ref[...] = s_sc[...]


def _sc_reduce_body(x_hbm, lg_hbm, seg_hbm, m_hbm, d_hbm, hist_hbm, out_hbm,
                    xb, lv, segv, wv, mv, dv, histv, accv, idxv, shacc, sem):
    cid = lax.axis_index("c")
    sid = lax.axis_index("s")
    wid = sid * NCORES + cid
    base = wid * TOK

    pltpu.sync_copy(lg_hbm.at[pl.ds(base, TOK)], lv)
    pltpu.sync_copy(seg_hbm.at[pl.ds(base, TOK)], segv)
    pltpu.sync_copy(m_hbm, mv)
    pltpu.sync_copy(d_hbm, dv)
    pltpu.sync_copy(hist_hbm.at[wid], histv)

    for i in range(TOK // 16):
        sl = pl.ds(i * 16, 16)
        sg = segv[sl]
        mm = plsc.load_gather(mv, [sg])
        dd = plsc.load_gather(dv, [sg])
        wv[sl] = jnp.exp(lv[sl] - mm) / (dd + 1e-16)

    hist_i = histv[...].astype(jnp.int32)
    ends = jnp.cumsum(hist_i)
    starts = ends - hist_i
    lane = lax.iota(jnp.int32, 16)
    los = [jnp.max(jnp.where(lane == s, starts, 0)) for s in range(B)]
    his = [jnp.max(jnp.where(lane == s, ends, 0)) for s in range(B)]

    zero16 = jnp.zeros((16,), jnp.float32)
    for r in range(B):
        for j in range(NV):
            accv[r, pl.ds(j * 16, 16)] = zero16

    for c in range(NSUB):
        pltpu.sync_copy(x_hbm.at[pl.ds(base + c * SUB, SUB)], xb)
        for s in range(B):
            lo = jnp.maximum(los[s], c * SUB)
            hi = jnp.minimum(his[s], (c + 1) * SUB)

            def body(t, carry, _c=c):
                wb = plsc.load_gather(wv, [jnp.zeros((16,), jnp.int32) + t])
                tl = t - _c * SUB
                return tuple(
                    carry[j] + wb * xb[tl, pl.ds(j * 16, 16)]
                    for j in range(NV))

            res = lax.fori_loop(lo, hi, body,
                                tuple(zero16 for _ in range(NV)))
            for j in range(NV):
                accv[s, pl.ds(j * 16, 16)] += res[j]

    idxv[...] = lane

    @pl.when(sid == 0)
    def _seed():
        pltpu.sync_copy(accv, shacc)

    plsc.subcore_barrier()

    @pl.when(sid != 0)
    def _add():
        pltpu.sync_copy(accv, shacc.at[idxv], add=True)

    plsc.subcore_barrier()

    @pl.when(sid == 0)
    def _out():
        pltpu.sync_copy(shacc, out_hbm.at[cid])


_sc_gather = pl.kernel(
    _sc_gather_body,
    out_type=jax.ShapeDtypeStruct((B, H), jnp.float32),
    mesh=plsc.VectorSubcoreMesh(core_axis_name="c", subcore_axis_name="s"),
    scratch_types=[
        pltpu.VMEM((B,), jnp.int32),
        pltpu.VMEM((B, H), jnp.float32),
        pltpu.SemaphoreType.DMA,
    ],
)


_sc_reduce = pl.kernel(
    _sc_reduce_body,
    out_type=jax.ShapeDtypeStruct((NCORES, B, H), jnp.float32),
    mesh=plsc.VectorSubcoreMesh(core_axis_name="c", subcore_axis_name="s"),
    scratch_types=[
        pltpu.VMEM((SUB, H), jnp.float32),
        pltpu.VMEM((TOK,), jnp.float32),
        pltpu.VMEM((TOK,), jnp.int32),
        pltpu.VMEM((TOK,), jnp.float32),
        pltpu.VMEM((B,), jnp.float32),
        pltpu.VMEM((B,), jnp.float32),
        pltpu.VMEM((B,), jnp.float32),
        pltpu.VMEM((B, H), jnp.float32),
        pltpu.VMEM((B,), jnp.int32),
        pltpu.VMEM_SHARED((B, H), jnp.float32),
        pltpu.SemaphoreType.DMA,
    ],
    compiler_params=pltpu.CompilerParams(needs_layout_passes=False),
)


@jax.jit
def kernel(x, batch, last_ixs, W1, b1, W2, b2, qw, qb):
    segi = batch.astype(jnp.int32)
    seg2 = segi.reshape(N, 1)
    b12 = (b1 + b2).reshape(1, H)
    qb2 = qb.reshape(1, 1)

    vi = _sc_gather(x, last_ixs.astype(jnp.int32))

    lg, m, s, hist = pl.pallas_call(
        _tc_body,
        grid=(NBLK,),
        in_specs=[
            pl.BlockSpec((BLK, H), lambda i: (i, 0)),
            pl.BlockSpec((BLK, 1), lambda i: (i, 0)),
            pl.BlockSpec((B, H), lambda i: (0, 0)),
            pl.BlockSpec((H, H), lambda i: (0, 0)),
            pl.BlockSpec((1, H), lambda i: (0, 0)),
            pl.BlockSpec((H, H), lambda i: (0, 0)),
            pl.BlockSpec((H, 1), lambda i: (0, 0)),
            pl.BlockSpec((1, 1), lambda i: (0, 0)),
        ],
        out_specs=[
            pl.BlockSpec((BLK, 1), lambda i: (i, 0)),
            pl.BlockSpec((1, B), lambda i: (0, 0)),
            pl.BlockSpec((1, B), lambda i: (0, 0)),
            pl.BlockSpec((1, 1, B), lambda i: (i, 0, 0)),
        ],
        out_shape=[
            jax.ShapeDtypeStruct((N, 1), jnp.float32),
            jax.ShapeDtypeStruct((1, B), jnp.float32),
            jax.ShapeDtypeStruct((1, B), jnp.float32),
            jax.ShapeDtypeStruct((NBLK, 1, B), jnp.float32),
        ],
        scratch_shapes=[
            pltpu.VMEM((B, H), jnp.float32),
            pltpu.VMEM((1, B), jnp.float32),
            pltpu.VMEM((1, B), jnp.float32),
        ],
        compiler_params=pltpu.CompilerParams(
            dimension_semantics=("arbitrary",),
        ),
    )(x, seg2, vi, W1, b12, W2, qw, qb2)

    partials = _sc_reduce(x, lg.reshape(N), segi, m.reshape(B),
                          s.reshape(B), hist.reshape(NBLK, B))
    return partials[0] + partials[1]


# speedup vs baseline: 1.6298x; 1.6298x over previous
import functools
import jax
import jax.numpy as jnp
from jax import lax
from jax.experimental import pallas as pl
from jax.experimental.pallas import tpu as pltpu
from jax.experimental.pallas import tpu_sc as plsc

B = 16
N = 32768
H = 128
BLK = 4096
NBLK = N // BLK
NCH = BLK // 1024
NCORES = 2
NTILES = 32
TOK = N // NTILES
SUB = 256
NSUB = TOK // SUB
NV = H // 16


def _tc_body(last_sm, x_any, x_ref, seg_ref, W1_ref, b12_ref, W2_ref, qw_ref,
             qb_ref, lg_ref, m_ref, s_ref, hist_ref, u_sc, s_sc, vi_sc, usm,
             sem):
    i = pl.program_id(0)

    @pl.when(i == 0)
    def _init():
        cps = [
            pltpu.make_async_copy(x_any.at[pl.ds(last_sm[b], 1)],
                                  vi_sc.at[pl.ds(b, 1)], sem)
            for b in range(B)
        ]
        for cp in cps:
            cp.start()
        for cp in cps:
            cp.wait()
        u_sc[...] = jnp.dot(vi_sc[...], W1_ref[...],
                            preferred_element_type=jnp.float32) + b12_ref[...]
        usm[0] = jnp.sum(jnp.maximum(qw_ref[...], 0.0)) + qb_ref[0, 0]
        s_sc[...] = jnp.zeros((B, 1), jnp.float32)

    x = x_ref[...]
    seg_row = seg_ref[0]
    iota = lax.broadcasted_iota(jnp.int32, (B, BLK), 0)
    ohT = (seg_row == iota).astype(jnp.float32)

    z = jnp.dot(x, W2_ref[...], preferred_element_type=jnp.float32)
    z = z + lax.dot_general(ohT, u_sc[...], (((0,), (0,)), ((), ())),
                            preferred_element_type=jnp.float32)
    h = 0.5 * jnp.tanh(0.5 * z) + 0.5
    lgr = lax.dot_general(qw_ref[...], h, (((0,), (1,)), ((), ())),
                          preferred_element_type=jnp.float32) + qb_ref[...]
    lg_ref[...] = lgr[None]

    hist_ref[...] = jnp.concatenate(
        [jnp.sum(ohT[:, c * 1024:(c + 1) * 1024], axis=1, keepdims=True)
         for c in range(NCH)], axis=1)[None]

    ex = jnp.exp(lgr - usm[0])
    s_sc[...] = s_sc[...] + jnp.sum(ohT * ex, axis=1, keepdims=True)

    @pl.when(i == NBLK - 1)
    def _fin():
        m_ref[...] = jnp.full((B, 1), usm[0], jnp.float32)
        s_ref[...] = s_sc[...]


def _sc_reduce_body(x_hbm, lg_hbm, seg_hbm, m_hbm, d_hbm, hist_hbm, out_hbm,
                    xb, lv, segv, wv, mv, dv, histv, accv, idxv, shacc, sem):
    cid = lax.axis_index("c")
    sid = lax.axis_index("s")
    wid = sid * NCORES + cid
    base = wid * TOK

    pltpu.sync_copy(lg_hbm.at[pl.ds(base, TOK)], lv)
    pltpu.sync_copy(seg_hbm.at[pl.ds(base, TOK)], segv)
    pltpu.sync_copy(m_hbm, mv)
    pltpu.sync_copy(d_hbm, dv)
    pltpu.sync_copy(hist_hbm.at[wid], histv)

    for i in range(TOK // 16):
        sl = pl.ds(i * 16, 16)
        sg = segv[sl]
        mm = plsc.load_gather(mv, [sg])
        dd = plsc.load_gather(dv, [sg])
        wv[sl] = jnp.exp(lv[sl] - mm) / (dd + 1e-16)

    hist_i = histv[...].astype(jnp.int32)
    ends = jnp.cumsum(hist_i)
    starts = ends - hist_i
    lane = lax.iota(jnp.int32, 16)
    los = [jnp.max(jnp.where(lane == s, starts, 0)) for s in range(B)]
    his = [jnp.max(jnp.where(lane == s, ends, 0)) for s in range(B)]

    zero16 = jnp.zeros((16,), jnp.float32)
    for r in range(B):
        for j in range(NV):
            accv[r, pl.ds(j * 16, 16)] = zero16

    for c in range(NSUB):
        pltpu.sync_copy(x_hbm.at[pl.ds(base + c * SUB, SUB)], xb)
        for s in range(B):
            lo = jnp.maximum(los[s], c * SUB)
            hi = jnp.minimum(his[s], (c + 1) * SUB)

            def body(t, carry, _c=c):
                wb = plsc.load_gather(wv, [jnp.zeros((16,), jnp.int32) + t])
                tl = t - _c * SUB
                return tuple(
                    carry[j] + wb * xb[tl, pl.ds(j * 16, 16)]
                    for j in range(NV))

            res = lax.fori_loop(lo, hi, body,
                                tuple(zero16 for _ in range(NV)))
            for j in range(NV):
                accv[s, pl.ds(j * 16, 16)] += res[j]

    idxv[...] = lane

    @pl.when(sid == 0)
    def _seed():
        pltpu.sync_copy(accv, shacc)

    plsc.subcore_barrier()

    @pl.when(sid != 0)
    def _add():
        pltpu.sync_copy(accv, shacc.at[idxv], add=True)

    plsc.subcore_barrier()

    @pl.when(sid == 0)
    def _out():
        pltpu.sync_copy(shacc, out_hbm.at[cid])


_sc_reduce = pl.kernel(
    _sc_reduce_body,
    out_type=jax.ShapeDtypeStruct((NCORES, B, H), jnp.float32),
    mesh=plsc.VectorSubcoreMesh(core_axis_name="c", subcore_axis_name="s"),
    scratch_types=[
        pltpu.VMEM((SUB, H), jnp.float32),
        pltpu.VMEM((TOK,), jnp.float32),
        pltpu.VMEM((TOK,), jnp.int32),
        pltpu.VMEM((TOK,), jnp.float32),
        pltpu.VMEM((B,), jnp.float32),
        pltpu.VMEM((B,), jnp.float32),
        pltpu.VMEM((B,), jnp.float32),
        pltpu.VMEM((B, H), jnp.float32),
        pltpu.VMEM((B,), jnp.int32),
        pltpu.VMEM_SHARED((B, H), jnp.float32),
        pltpu.SemaphoreType.DMA,
    ],
    compiler_params=pltpu.CompilerParams(needs_layout_passes=False),
)


@jax.jit
def kernel(x, batch, last_ixs, W1, b1, W2, b2, qw, qb):
    segi = batch.astype(jnp.int32)
    seg3 = segi.reshape(NBLK, 1, BLK)
    b12 = (b1 + b2).reshape(1, H)
    qb2 = qb.reshape(1, 1)

    lg, m, s, hist = pl.pallas_call(
        _tc_body,
        grid=(NBLK,),
        in_specs=[
            pl.BlockSpec(memory_space=pltpu.SMEM),
            pl.BlockSpec(memory_space=pltpu.MemorySpace.HBM),
            pl.BlockSpec((BLK, H), lambda i: (i, 0)),
            pl.BlockSpec((1, 1, BLK), lambda i: (i, 0, 0)),
            pl.BlockSpec((H, H), lambda i: (0, 0)),
            pl.BlockSpec((1, H), lambda i: (0, 0)),
            pl.BlockSpec((H, H), lambda i: (0, 0)),
            pl.BlockSpec((H, 1), lambda i: (0, 0)),
            pl.BlockSpec((1, 1), lambda i: (0, 0)),
        ],
        out_specs=[
            pl.BlockSpec((1, 1, BLK), lambda i: (i, 0, 0)),
            pl.BlockSpec((B, 1), lambda i: (0, 0)),
            pl.BlockSpec((B, 1), lambda i: (0, 0)),
            pl.BlockSpec((1, B, NCH), lambda i: (i, 0, 0)),
        ],
        out_shape=[
            jax.ShapeDtypeStruct((NBLK, 1, BLK), jnp.float32),
            jax.ShapeDtypeStruct((B, 1), jnp.float32),
            jax.ShapeDtypeStruct((B, 1), jnp.float32),
            jax.ShapeDtypeStruct((NBLK, B, NCH), jnp.float32),
        ],
        scratch_shapes=[
            pltpu.VMEM((B, H), jnp.float32),
            pltpu.VMEM((B, 1), jnp.float32),
            pltpu.VMEM((B, H), jnp.float32),
            pltpu.SMEM((1,), jnp.float32),
            pltpu.SemaphoreType.DMA,
        ],
        compiler_params=pltpu.CompilerParams(
            dimension_semantics=("arbitrary",),
            fuse_transposed_lhs_in_matmul=True,
        ),
    )(last_ixs.astype(jnp.int32), x, x, seg3, W1, b12, W2, qw, qb2)

    hist32 = jnp.transpose(hist, (0, 2, 1)).reshape(NTILES, B)
    partials = _sc_reduce(x, lg.reshape(N), segi, m.reshape(B),
                          s.reshape(B), hist32)
    return partials[0] + partials[1]


# speedup vs baseline: 1.9610x; 1.2032x over previous
import functools
import jax
import jax.numpy as jnp
from jax import lax
from jax.experimental import pallas as pl
from jax.experimental.pallas import tpu as pltpu
from jax.experimental.pallas import tpu_sc as plsc

B = 16
N = 32768
H = 128
BLK = 4096
NBLK = N // BLK
NCH = BLK // 1024
NCORES = 2
NTILES = 32
TOK = N // NTILES
SUB = 256
NSUB = TOK // SUB
NV = H // 16


def _tc_body(last_sm, x_any, x_ref, seg_ref, W1_ref, b12_ref, W2_ref, qw_ref,
             qb_ref, lg_ref, m_ref, s_ref, hist_ref, u_sc, s_sc, vi_sc, usm,
             sem):
    i = pl.program_id(0)

    @pl.when(i == 0)
    def _init():
        cps = [
            pltpu.make_async_copy(x_any.at[pl.ds(last_sm[b], 1)],
                                  vi_sc.at[pl.ds(b, 1)], sem)
            for b in range(B)
        ]
        for cp in cps:
            cp.start()
        for cp in cps:
            cp.wait()
        u_sc[...] = jnp.dot(vi_sc[...], W1_ref[...],
                            preferred_element_type=jnp.float32) + b12_ref[...]
        usm[0] = jnp.sum(jnp.maximum(qw_ref[...], 0.0)) + qb_ref[0, 0]
        s_sc[...] = jnp.zeros((B, 1), jnp.float32)

    x = x_ref[...]
    seg_row = seg_ref[0]
    iota = lax.broadcasted_iota(jnp.int32, (B, BLK), 0)
    ohT = (seg_row == iota).astype(jnp.float32)

    z = jnp.dot(x, W2_ref[...], preferred_element_type=jnp.float32)
    z = z + lax.dot_general(ohT, u_sc[...], (((0,), (0,)), ((), ())),
                            preferred_element_type=jnp.float32)
    h = 0.5 * jnp.tanh(0.5 * z) + 0.5
    lgr = lax.dot_general(qw_ref[...], h, (((0,), (1,)), ((), ())),
                          preferred_element_type=jnp.float32) + qb_ref[...]
    lg_ref[...] = lgr[None]

    hist_ref[...] = jnp.concatenate(
        [jnp.sum(ohT[:, c * 1024:(c + 1) * 1024], axis=1, keepdims=True)
         for c in range(NCH)], axis=1)[None]

    ex = jnp.exp(lgr - usm[0])
    s_sc[...] = s_sc[...] + jnp.sum(ohT * ex, axis=1, keepdims=True)

    @pl.when(i == NBLK - 1)
    def _fin():
        m_ref[...] = jnp.full((B, 1), usm[0], jnp.float32)
        s_ref[...] = s_sc[...]


def _sc_reduce_body(x_hbm, lg_hbm, seg_hbm, m_hbm, d_hbm, hist_hbm, out_hbm,
                    xb, lv, segv, wv, mv, dv, histv, accv, idxv, shacc,
                    sem0, sem1):
    cid = lax.axis_index("c")
    sid = lax.axis_index("s")
    wid = sid * NCORES + cid
    base = wid * TOK
    sems = [sem0, sem1]

    cps = [None] * NSUB
    for c in range(min(2, NSUB)):
        cps[c] = pltpu.async_copy(x_hbm.at[pl.ds(base + c * SUB, SUB)],
                                  xb.at[c % 2], sems[c % 2])

    pltpu.sync_copy(lg_hbm.at[pl.ds(base, TOK)], lv)
    pltpu.sync_copy(seg_hbm.at[pl.ds(base, TOK)], segv)
    pltpu.sync_copy(m_hbm, mv)
    pltpu.sync_copy(d_hbm, dv)
    pltpu.sync_copy(hist_hbm.at[wid], histv)

    @plsc.parallel_loop(0, TOK // 16, unroll=4)
    def _w(i):
        sl = pl.ds(i * 16, 16)
        sg = segv[sl]
        mm = plsc.load_gather(mv, [sg])
        dd = plsc.load_gather(dv, [sg])
        wv[sl] = jnp.exp(lv[sl] - mm) / (dd + 1e-16)

    hist_i = histv[...].astype(jnp.int32)
    ends = jnp.cumsum(hist_i)
    starts = ends - hist_i
    lane = lax.iota(jnp.int32, 16)

    zero16 = jnp.zeros((16,), jnp.float32)
    for r in range(B):
        for j in range(NV):
            accv[r, pl.ds(j * 16, 16)] = zero16

    for c in range(NSUB):
        cps[c].wait()

        def sbody(s, _, _c=c):
            lo = jnp.maximum(jnp.max(jnp.where(lane == s, starts, 0)),
                             _c * SUB)
            hi = jnp.minimum(jnp.max(jnp.where(lane == s, ends, 0)),
                             (_c + 1) * SUB)

            @plsc.parallel_loop(lo, hi, unroll=4,
                                carry=tuple(zero16 for _ in range(NV)))
            def res(t, carry):
                wb = plsc.load_gather(wv, [jnp.zeros((16,), jnp.int32) + t])
                tl = t - _c * SUB
                return tuple(
                    carry[j] + wb * xb[_c % 2, tl, pl.ds(j * 16, 16)]
                    for j in range(NV))

            for j in range(NV):
                accv[s, pl.ds(j * 16, 16)] += res[j]
            return 0

        lax.fori_loop(0, B, sbody, 0)
        if c + 2 < NSUB:
            cps[c + 2] = pltpu.async_copy(
                x_hbm.at[pl.ds(base + (c + 2) * SUB, SUB)],
                xb.at[c % 2], sems[c % 2])

    idxv[...] = lane

    @pl.when(sid == 0)
    def _seed():
        pltpu.sync_copy(accv, shacc)

    plsc.subcore_barrier()

    @pl.when(sid != 0)
    def _add():
        pltpu.sync_copy(accv, shacc.at[idxv], add=True)

    plsc.subcore_barrier()

    @pl.when(sid == 0)
    def _out():
        pltpu.sync_copy(shacc, out_hbm.at[cid])


_sc_reduce = pl.kernel(
    _sc_reduce_body,
    out_type=jax.ShapeDtypeStruct((NCORES, B, H), jnp.float32),
    mesh=plsc.VectorSubcoreMesh(core_axis_name="c", subcore_axis_name="s"),
    scratch_types=[
        pltpu.VMEM((2, SUB, H), jnp.float32),
        pltpu.VMEM((TOK,), jnp.float32),
        pltpu.VMEM((TOK,), jnp.int32),
        pltpu.VMEM((TOK,), jnp.float32),
        pltpu.VMEM((B,), jnp.float32),
        pltpu.VMEM((B,), jnp.float32),
        pltpu.VMEM((B,), jnp.float32),
        pltpu.VMEM((B, H), jnp.float32),
        pltpu.VMEM((B,), jnp.int32),
        pltpu.VMEM_SHARED((B, H), jnp.float32),
        pltpu.SemaphoreType.DMA,
        pltpu.SemaphoreType.DMA,
    ],
    compiler_params=pltpu.CompilerParams(needs_layout_passes=False),
)


@jax.jit
def kernel(x, batch, last_ixs, W1, b1, W2, b2, qw, qb):
    segi = batch.astype(jnp.int32)
    seg3 = segi.reshape(NBLK, 1, BLK)
    b12 = (b1 + b2).reshape(1, H)
    qb2 = qb.reshape(1, 1)

    lg, m, s, hist = pl.pallas_call(
        _tc_body,
        grid=(NBLK,),
        in_specs=[
            pl.BlockSpec(memory_space=pltpu.SMEM),
            pl.BlockSpec(memory_space=pltpu.MemorySpace.HBM),
            pl.BlockSpec((BLK, H), lambda i: (i, 0)),
            pl.BlockSpec((1, 1, BLK), lambda i: (i, 0, 0)),
            pl.BlockSpec((H, H), lambda i: (0, 0)),
            pl.BlockSpec((1, H), lambda i: (0, 0)),
            pl.BlockSpec((H, H), lambda i: (0, 0)),
            pl.BlockSpec((H, 1), lambda i: (0, 0)),
            pl.BlockSpec((1, 1), lambda i: (0, 0)),
        ],
        out_specs=[
            pl.BlockSpec((1, 1, BLK), lambda i: (i, 0, 0)),
            pl.BlockSpec((B, 1), lambda i: (0, 0)),
            pl.BlockSpec((B, 1), lambda i: (0, 0)),
            pl.BlockSpec((1, B, NCH), lambda i: (i, 0, 0)),
        ],
        out_shape=[
            jax.ShapeDtypeStruct((NBLK, 1, BLK), jnp.float32),
            jax.ShapeDtypeStruct((B, 1), jnp.float32),
            jax.ShapeDtypeStruct((B, 1), jnp.float32),
            jax.ShapeDtypeStruct((NBLK, B, NCH), jnp.float32),
        ],
        scratch_shapes=[
            pltpu.VMEM((B, H), jnp.float32),
            pltpu.VMEM((B, 1), jnp.float32),
            pltpu.VMEM((B, H), jnp.float32),
            pltpu.SMEM((1,), jnp.float32),
            pltpu.SemaphoreType.DMA,
        ],
        compiler_params=pltpu.CompilerParams(
            dimension_semantics=("arbitrary",),
            fuse_transposed_lhs_in_matmul=True,
        ),
    )(last_ixs.astype(jnp.int32), x, x, seg3, W1, b12, W2, qw, qb2)

    hist32 = jnp.transpose(hist, (0, 2, 1)).reshape(NTILES, B)
    partials = _sc_reduce(x, lg.reshape(N), segi, m.reshape(B),
                          s.reshape(B), hist32)
    return partials[0] + partials[1]


# speedup vs baseline: 2.0636x; 1.0523x over previous
import functools
import jax
import jax.numpy as jnp
from jax import lax
from jax.experimental import pallas as pl
from jax.experimental.pallas import tpu as pltpu
from jax.experimental.pallas import tpu_sc as plsc

B = 16
N = 32768
H = 128
BLK = 4096
NBLK = N // BLK
NCH = BLK // 1024
NCORES = 2
NTILES = 32
TOK = N // NTILES
SUB = 256
NSUB = TOK // SUB
NV = H // 16


def _tc_body(last_sm, x_any, x_ref, seg_ref, W1_ref, b12_ref, W2_ref, qw_ref,
             qb_ref, lg_ref, m_ref, s_ref, hist_ref, u_sc, s_sc, vi_sc, usm,
             sem):
    i = pl.program_id(0)

    @pl.when(i == 0)
    def _init():
        cps = [
            pltpu.make_async_copy(x_any.at[pl.ds(last_sm[b], 1)],
                                  vi_sc.at[pl.ds(b, 1)], sem)
            for b in range(B)
        ]
        for cp in cps:
            cp.start()
        for cp in cps:
            cp.wait()
        u_sc[...] = jnp.dot(vi_sc[...], W1_ref[...],
                            preferred_element_type=jnp.float32) + b12_ref[...]
        usm[0] = jnp.sum(jnp.maximum(qw_ref[...], 0.0)) + qb_ref[0, 0]
        s_sc[...] = jnp.zeros((B, 1), jnp.float32)

    x = x_ref[...]
    seg_row = seg_ref[0]
    iota = lax.broadcasted_iota(jnp.int32, (B, BLK), 0)
    ohT = (seg_row == iota).astype(jnp.float32)

    z = jnp.dot(x, W2_ref[...], preferred_element_type=jnp.float32)
    z = z + lax.dot_general(ohT, u_sc[...], (((0,), (0,)), ((), ())),
                            preferred_element_type=jnp.float32)
    h = 0.5 * jnp.tanh(0.5 * z) + 0.5
    lgr = lax.dot_general(qw_ref[...], h, (((0,), (1,)), ((), ())),
                          preferred_element_type=jnp.float32) + qb_ref[...]
    lg_ref[...] = lgr[None]

    ones_row = jnp.ones((1, 1024), jnp.float32)
    hist_ref[...] = jnp.concatenate(
        [lax.dot_general(ones_row, ohT[:, c * 1024:(c + 1) * 1024],
                         (((1,), (1,)), ((), ())),
                         preferred_element_type=jnp.float32)[None]
         for c in range(NCH)], axis=0)

    ex = jnp.exp(lgr - usm[0])
    s_sc[...] = s_sc[...] + jnp.sum(ohT * ex, axis=1, keepdims=True)

    @pl.when(i == NBLK - 1)
    def _fin():
        m_ref[...] = jnp.full((B, 1), usm[0], jnp.float32)
        s_ref[...] = s_sc[...]


def _sc_reduce_body(x_hbm, lg_hbm, seg_hbm, m_hbm, d_hbm, hist_hbm, out_hbm,
                    xb, lv, segv, wv, mv, dv, histv, accv, idxv, shacc,
                    sem0, sem1):
    cid = lax.axis_index("c")
    sid = lax.axis_index("s")
    wid = sid * NCORES + cid
    base = wid * TOK
    sems = [sem0, sem1]

    cps = [None] * NSUB
    for c in range(min(2, NSUB)):
        cps[c] = pltpu.async_copy(x_hbm.at[pl.ds(base + c * SUB, SUB)],
                                  xb.at[c % 2], sems[c % 2])

    pltpu.sync_copy(lg_hbm.at[pl.ds(base, TOK)], lv)
    pltpu.sync_copy(seg_hbm.at[pl.ds(base, TOK)], segv)
    pltpu.sync_copy(m_hbm, mv)
    pltpu.sync_copy(d_hbm, dv)
    pltpu.sync_copy(hist_hbm.at[wid], histv)

    @plsc.parallel_loop(0, TOK // 16, unroll=4)
    def _w(i):
        sl = pl.ds(i * 16, 16)
        sg = segv[sl]
        mm = plsc.load_gather(mv, [sg])
        dd = plsc.load_gather(dv, [sg])
        wv[sl] = jnp.exp(lv[sl] - mm) / (dd + 1e-16)

    hist_i = histv[...].astype(jnp.int32)
    ends = jnp.cumsum(hist_i)
    starts = ends - hist_i
    lane = lax.iota(jnp.int32, 16)
    s_first = jnp.min(jnp.where(hist_i > 0, lane, B))
    s_last = jnp.max(jnp.where(hist_i > 0, lane, -1))

    zero16 = jnp.zeros((16,), jnp.float32)
    for r in range(B):
        for j in range(NV):
            accv[r, pl.ds(j * 16, 16)] = zero16

    for c in range(NSUB):
        cps[c].wait()

        def sbody(s, _, _c=c):
            lo = jnp.maximum(jnp.max(jnp.where(lane == s, starts, 0)),
                             _c * SUB)
            hi = jnp.minimum(jnp.max(jnp.where(lane == s, ends, 0)),
                             (_c + 1) * SUB)

            @plsc.parallel_loop(lo, hi, unroll=8,
                                carry=tuple(zero16 for _ in range(NV)))
            def res(t, carry):
                wb = plsc.load_gather(wv, [jnp.zeros((16,), jnp.int32) + t])
                tl = t - _c * SUB
                return tuple(
                    carry[j] + wb * xb[_c % 2, tl, pl.ds(j * 16, 16)]
                    for j in range(NV))

            for j in range(NV):
                accv[s, pl.ds(j * 16, 16)] += res[j]
            return 0

        lax.fori_loop(s_first, s_last + 1, sbody, 0)
        if c + 2 < NSUB:
            cps[c + 2] = pltpu.async_copy(
                x_hbm.at[pl.ds(base + (c + 2) * SUB, SUB)],
                xb.at[c % 2], sems[c % 2])

    idxv[...] = lane

    @pl.when(sid == 0)
    def _seed():
        pltpu.sync_copy(accv, shacc)

    plsc.subcore_barrier()

    @pl.when(sid != 0)
    def _add():
        pltpu.sync_copy(accv, shacc.at[idxv], add=True)

    plsc.subcore_barrier()

    @pl.when(sid == 0)
    def _out():
        pltpu.sync_copy(shacc, out_hbm.at[cid])


_sc_reduce = pl.kernel(
    _sc_reduce_body,
    out_type=jax.ShapeDtypeStruct((NCORES, B, H), jnp.float32),
    mesh=plsc.VectorSubcoreMesh(core_axis_name="c", subcore_axis_name="s"),
    scratch_types=[
        pltpu.VMEM((2, SUB, H), jnp.float32),
        pltpu.VMEM((TOK,), jnp.float32),
        pltpu.VMEM((TOK,), jnp.int32),
        pltpu.VMEM((TOK,), jnp.float32),
        pltpu.VMEM((B,), jnp.float32),
        pltpu.VMEM((B,), jnp.float32),
        pltpu.VMEM((B,), jnp.float32),
        pltpu.VMEM((B, H), jnp.float32),
        pltpu.VMEM((B,), jnp.int32),
        pltpu.VMEM_SHARED((B, H), jnp.float32),
        pltpu.SemaphoreType.DMA,
        pltpu.SemaphoreType.DMA,
    ],
    compiler_params=pltpu.CompilerParams(needs_layout_passes=False),
)


@jax.jit
def kernel(x, batch, last_ixs, W1, b1, W2, b2, qw, qb):
    segi = batch.astype(jnp.int32)
    seg3 = segi.reshape(NBLK, 1, BLK)
    b12 = (b1 + b2).reshape(1, H)
    qb2 = qb.reshape(1, 1)

    lg, m, s, hist = pl.pallas_call(
        _tc_body,
        grid=(NBLK,),
        in_specs=[
            pl.BlockSpec(memory_space=pltpu.SMEM),
            pl.BlockSpec(memory_space=pltpu.MemorySpace.HBM),
            pl.BlockSpec((BLK, H), lambda i: (i, 0)),
            pl.BlockSpec((1, 1, BLK), lambda i: (i, 0, 0)),
            pl.BlockSpec((H, H), lambda i: (0, 0)),
            pl.BlockSpec((1, H), lambda i: (0, 0)),
            pl.BlockSpec((H, H), lambda i: (0, 0)),
            pl.BlockSpec((H, 1), lambda i: (0, 0)),
            pl.BlockSpec((1, 1), lambda i: (0, 0)),
        ],
        out_specs=[
            pl.BlockSpec((1, 1, BLK), lambda i: (i, 0, 0)),
            pl.BlockSpec((B, 1), lambda i: (0, 0)),
            pl.BlockSpec((B, 1), lambda i: (0, 0)),
            pl.BlockSpec((NCH, 1, B), lambda i: (i, 0, 0)),
        ],
        out_shape=[
            jax.ShapeDtypeStruct((NBLK, 1, BLK), jnp.float32),
            jax.ShapeDtypeStruct((B, 1), jnp.float32),
            jax.ShapeDtypeStruct((B, 1), jnp.float32),
            jax.ShapeDtypeStruct((NTILES, 1, B), jnp.float32),
        ],
        scratch_shapes=[
            pltpu.VMEM((B, H), jnp.float32),
            pltpu.VMEM((B, 1), jnp.float32),
            pltpu.VMEM((B, H), jnp.float32),
            pltpu.SMEM((1,), jnp.float32),
            pltpu.SemaphoreType.DMA,
        ],
        compiler_params=pltpu.CompilerParams(
            dimension_semantics=("arbitrary",),
            fuse_transposed_lhs_in_matmul=True,
        ),
    )(last_ixs.astype(jnp.int32), x, x, seg3, W1, b12, W2, qw, qb2)

    partials = _sc_reduce(x, lg.reshape(N), segi, m.reshape(B),
                          s.reshape(B), hist.reshape(NTILES, B))
    return partials[0] + partials[1]


# speedup vs baseline: 2.1688x; 1.0510x over previous
import functools
import jax
import jax.numpy as jnp
from jax import lax
from jax.experimental import pallas as pl
from jax.experimental.pallas import tpu as pltpu
from jax.experimental.pallas import tpu_sc as plsc

B = 16
N = 32768
H = 128
BLK = 8192
NBLK = N // BLK
NCH = BLK // 1024
NCORES = 2
NTILES = 32
TOK = N // NTILES
SUB = 256
NSUB = TOK // SUB
NV = H // 16


def _tc_body(last_sm, x_any, x_ref, seg_ref, W1_ref, b12_ref, W2_ref, qw_ref,
             qb_ref, lg_ref, m_ref, s_ref, hist_ref, u_sc, s_sc, vi_sc, usm,
             sem):
    i = pl.program_id(0)

    @pl.when(i == 0)
    def _init():
        cps = [
            pltpu.make_async_copy(x_any.at[pl.ds(last_sm[b], 1)],
                                  vi_sc.at[pl.ds(b, 1)], sem)
            for b in range(B)
        ]
        for cp in cps:
            cp.start()
        for cp in cps:
            cp.wait()
        u_sc[...] = jnp.dot(vi_sc[...], W1_ref[...],
                            preferred_element_type=jnp.float32) + b12_ref[...]
        usm[0] = jnp.sum(jnp.maximum(qw_ref[...], 0.0)) + qb_ref[0, 0]
        s_sc[...] = jnp.zeros((B, 1), jnp.float32)

    x = x_ref[...]
    seg_row = seg_ref[0]
    iota = lax.broadcasted_iota(jnp.int32, (B, BLK), 0)
    ohT = (seg_row == iota).astype(jnp.float32)

    z = jnp.dot(x, W2_ref[...], preferred_element_type=jnp.float32)
    z = z + lax.dot_general(ohT, u_sc[...], (((0,), (0,)), ((), ())),
                            preferred_element_type=jnp.float32)
    h = 0.5 * jnp.tanh(0.5 * z) + 0.5
    lgr = lax.dot_general(qw_ref[...], h, (((0,), (1,)), ((), ())),
                          preferred_element_type=jnp.float32) + qb_ref[...]
    lg_ref[...] = lgr[None]

    ones_row = jnp.ones((1, 1024), jnp.float32)
    hist_ref[...] = jnp.concatenate(
        [lax.dot_general(ones_row, ohT[:, c * 1024:(c + 1) * 1024],
                         (((1,), (1,)), ((), ())),
                         preferred_element_type=jnp.float32)[None]
         for c in range(NCH)], axis=0)

    ex = jnp.exp(lgr - usm[0])
    s_sc[...] = s_sc[...] + jnp.sum(ohT * ex, axis=1, keepdims=True)

    @pl.when(i == NBLK - 1)
    def _fin():
        m_ref[...] = jnp.full((B, 1), usm[0], jnp.float32)
        s_ref[...] = s_sc[...]


def _sc_reduce_body(x_hbm, lg_hbm, seg_hbm, m_hbm, d_hbm, hist_hbm, out_hbm,
                    xb, lv, segv, wv, mv, dv, histv, accv, idxv, shacc,
                    sem0, sem1, semh):
    cid = lax.axis_index("c")
    sid = lax.axis_index("s")
    wid = sid * NCORES + cid
    base = wid * TOK
    sems = [sem0, sem1]

    cps = [None] * NSUB
    for c in range(min(2, NSUB)):
        cps[c] = pltpu.async_copy(x_hbm.at[pl.ds(base + c * SUB, SUB)],
                                  xb.at[c % 2], sems[c % 2])

    hdr = [
        pltpu.async_copy(lg_hbm.at[pl.ds(base, TOK)], lv, semh),
        pltpu.async_copy(seg_hbm.at[pl.ds(base, TOK)], segv, semh),
        pltpu.async_copy(m_hbm, mv, semh),
        pltpu.async_copy(d_hbm, dv, semh),
        pltpu.async_copy(hist_hbm.at[wid], histv, semh),
    ]
    for cp in hdr:
        cp.wait()

    @plsc.parallel_loop(0, TOK // 16, unroll=4)
    def _w(i):
        sl = pl.ds(i * 16, 16)
        sg = segv[sl]
        mm = plsc.load_gather(mv, [sg])
        dd = plsc.load_gather(dv, [sg])
        wv[sl] = jnp.exp(lv[sl] - mm) / (dd + 1e-16)

    hist_i = histv[...].astype(jnp.int32)
    ends = jnp.cumsum(hist_i)
    starts = ends - hist_i
    lane = lax.iota(jnp.int32, 16)
    s_first = jnp.min(jnp.where(hist_i > 0, lane, B))
    s_last = jnp.max(jnp.where(hist_i > 0, lane, -1))

    zero16 = jnp.zeros((16,), jnp.float32)
    for r in range(B):
        for j in range(NV):
            accv[r, pl.ds(j * 16, 16)] = zero16

    for c in range(NSUB):
        cps[c].wait()

        def sbody(s, _, _c=c):
            lo = jnp.maximum(jnp.max(jnp.where(lane == s, starts, 0)),
                             _c * SUB)
            hi = jnp.minimum(jnp.max(jnp.where(lane == s, ends, 0)),
                             (_c + 1) * SUB)

            @plsc.parallel_loop(lo, hi, unroll=8,
                                carry=tuple(zero16 for _ in range(NV)))
            def res(t, carry):
                wb = plsc.load_gather(wv, [jnp.zeros((16,), jnp.int32) + t])
                tl = t - _c * SUB
                return tuple(
                    carry[j] + wb * xb[_c % 2, tl, pl.ds(j * 16, 16)]
                    for j in range(NV))

            for j in range(NV):
                accv[s, pl.ds(j * 16, 16)] += res[j]
            return 0

        lax.fori_loop(s_first, s_last + 1, sbody, 0)
        if c + 2 < NSUB:
            cps[c + 2] = pltpu.async_copy(
                x_hbm.at[pl.ds(base + (c + 2) * SUB, SUB)],
                xb.at[c % 2], sems[c % 2])

    idxv[...] = lane

    @pl.when(sid == 0)
    def _seed():
        pltpu.sync_copy(accv, shacc)

    plsc.subcore_barrier()

    @pl.when(sid != 0)
    def _add():
        pltpu.sync_copy(accv, shacc.at[idxv], add=True)

    plsc.subcore_barrier()

    @pl.when(sid == 0)
    def _out():
        pltpu.sync_copy(shacc, out_hbm.at[cid])


_sc_reduce = pl.kernel(
    _sc_reduce_body,
    out_type=jax.ShapeDtypeStruct((NCORES, B, H), jnp.float32),
    mesh=plsc.VectorSubcoreMesh(core_axis_name="c", subcore_axis_name="s"),
    scratch_types=[
        pltpu.VMEM((2, SUB, H), jnp.float32),
        pltpu.VMEM((TOK,), jnp.float32),
        pltpu.VMEM((TOK,), jnp.int32),
        pltpu.VMEM((TOK,), jnp.float32),
        pltpu.VMEM((B,), jnp.float32),
        pltpu.VMEM((B,), jnp.float32),
        pltpu.VMEM((B,), jnp.float32),
        pltpu.VMEM((B, H), jnp.float32),
        pltpu.VMEM((B,), jnp.int32),
        pltpu.VMEM_SHARED((B, H), jnp.float32),
        pltpu.SemaphoreType.DMA,
        pltpu.SemaphoreType.DMA,
        pltpu.SemaphoreType.DMA,
    ],
    compiler_params=pltpu.CompilerParams(needs_layout_passes=False),
)


@jax.jit
def kernel(x, batch, last_ixs, W1, b1, W2, b2, qw, qb):
    segi = batch.astype(jnp.int32)
    seg3 = segi.reshape(NBLK, 1, BLK)
    b12 = (b1 + b2).reshape(1, H)
    qb2 = qb.reshape(1, 1)

    lg, m, s, hist = pl.pallas_call(
        _tc_body,
        grid=(NBLK,),
        in_specs=[
            pl.BlockSpec(memory_space=pltpu.SMEM),
            pl.BlockSpec(memory_space=pltpu.MemorySpace.HBM),
            pl.BlockSpec((BLK, H), lambda i: (i, 0)),
            pl.BlockSpec((1, 1, BLK), lambda i: (i, 0, 0)),
            pl.BlockSpec((H, H), lambda i: (0, 0)),
            pl.BlockSpec((1, H), lambda i: (0, 0)),
            pl.BlockSpec((H, H), lambda i: (0, 0)),
            pl.BlockSpec((H, 1), lambda i: (0, 0)),
            pl.BlockSpec((1, 1), lambda i: (0, 0)),
        ],
        out_specs=[
            pl.BlockSpec((1, 1, BLK), lambda i: (i, 0, 0)),
            pl.BlockSpec((B, 1), lambda i: (0, 0)),
            pl.BlockSpec((B, 1), lambda i: (0, 0)),
            pl.BlockSpec((NCH, 1, B), lambda i: (i, 0, 0)),
        ],
        out_shape=[
            jax.ShapeDtypeStruct((NBLK, 1, BLK), jnp.float32),
            jax.ShapeDtypeStruct((B, 1), jnp.float32),
            jax.ShapeDtypeStruct((B, 1), jnp.float32),
            jax.ShapeDtypeStruct((NTILES, 1, B), jnp.float32),
        ],
        scratch_shapes=[
            pltpu.VMEM((B, H), jnp.float32),
            pltpu.VMEM((B, 1), jnp.float32),
            pltpu.VMEM((B, H), jnp.float32),
            pltpu.SMEM((1,), jnp.float32),
            pltpu.SemaphoreType.DMA,
        ],
        compiler_params=pltpu.CompilerParams(
            dimension_semantics=("arbitrary",),
            fuse_transposed_lhs_in_matmul=True,
        ),
    )(last_ixs.astype(jnp.int32), x, x, seg3, W1, b12, W2, qw, qb2)

    partials = _sc_reduce(x, lg.reshape(N), segi, m.reshape(B),
                          s.reshape(B), hist.reshape(NTILES, B))
    return partials[0] + partials[1]


# speedup vs baseline: 2.2139x; 1.0208x over previous
import functools
import jax
import jax.numpy as jnp
from jax import lax
from jax.experimental import pallas as pl
from jax.experimental.pallas import tpu as pltpu
from jax.experimental.pallas import tpu_sc as plsc

B = 16
N = 32768
H = 128
BLK = 8192
NBLK = N // BLK
NCH = BLK // 1024
NCORES = 2
NTILES = 32
TOK = N // NTILES
SUB = 256
NSUB = TOK // SUB
NV = H // 16


def _tc_body(last_sm, x_any, x_ref, seg_ref, W1_ref, b12_ref, W2_ref, qw_ref,
             qb_ref, lg_ref, s_ref, hist_ref, u_sc, s_sc, vi_sc, usm,
             sem):
    i = pl.program_id(0)

    @pl.when(i == 0)
    def _init():
        cps = [
            pltpu.make_async_copy(x_any.at[pl.ds(last_sm[b], 1)],
                                  vi_sc.at[pl.ds(b, 1)], sem)
            for b in range(B)
        ]
        for cp in cps:
            cp.start()
        for cp in cps:
            cp.wait()
        u_sc[...] = jnp.dot(vi_sc[...], W1_ref[...],
                            preferred_element_type=jnp.float32) + b12_ref[...]
        usm[0] = jnp.sum(jnp.maximum(qw_ref[...], 0.0)) + qb_ref[0, 0]
        s_sc[...] = jnp.zeros((B, 1), jnp.float32)

    x = x_ref[...]
    seg_row = seg_ref[0]
    iota = lax.broadcasted_iota(jnp.int32, (B, BLK), 0)
    ohT = (seg_row == iota).astype(jnp.float32)

    z = jnp.dot(x, W2_ref[...], preferred_element_type=jnp.float32)
    z = z + lax.dot_general(ohT, u_sc[...], (((0,), (0,)), ((), ())),
                            preferred_element_type=jnp.float32)
    h = 0.5 * jnp.tanh(0.5 * z) + 0.5
    lgr = lax.dot_general(qw_ref[...], h, (((0,), (1,)), ((), ())),
                          preferred_element_type=jnp.float32) + qb_ref[...]

    ones_row = jnp.ones((1, 1024), jnp.float32)
    hist_ref[...] = jnp.concatenate(
        [lax.dot_general(ones_row, ohT[:, c * 1024:(c + 1) * 1024],
                         (((1,), (1,)), ((), ())),
                         preferred_element_type=jnp.float32)[None]
         for c in range(NCH)], axis=0)

    ex = jnp.exp(lgr - usm[0])
    lg_ref[...] = ex[None]
    s_sc[...] = s_sc[...] + jnp.sum(ohT * ex, axis=1, keepdims=True)

    @pl.when(i == NBLK - 1)
    def _fin():
        s_ref[...] = s_sc[...]


def _sc_reduce_body(x_hbm, lg_hbm, seg_hbm, d_hbm, hist_hbm, out_hbm,
                    xb, lv, segv, dv, histv, accv, idxv, shacc,
                    sem0, sem1, semh):
    cid = lax.axis_index("c")
    sid = lax.axis_index("s")
    wid = sid * NCORES + cid
    base = wid * TOK
    sems = [sem0, sem1]

    cps = [None] * NSUB
    for c in range(min(2, NSUB)):
        cps[c] = pltpu.async_copy(x_hbm.at[pl.ds(base + c * SUB, SUB)],
                                  xb.at[c % 2], sems[c % 2])

    hdr = [
        pltpu.async_copy(lg_hbm.at[pl.ds(base, TOK)], lv, semh),
        pltpu.async_copy(seg_hbm.at[pl.ds(base, TOK)], segv, semh),
        pltpu.async_copy(d_hbm, dv, semh),
        pltpu.async_copy(hist_hbm.at[wid], histv, semh),
    ]
    for cp in hdr:
        cp.wait()

    hist_i = histv[...].astype(jnp.int32)
    ends = jnp.cumsum(hist_i)
    starts = ends - hist_i
    lane = lax.iota(jnp.int32, 16)
    s_first = jnp.min(jnp.where(hist_i > 0, lane, B))
    s_last = jnp.max(jnp.where(hist_i > 0, lane, -1))

    zero16 = jnp.zeros((16,), jnp.float32)
    for r in range(B):
        for j in range(NV):
            accv[r, pl.ds(j * 16, 16)] = zero16

    for c in range(NSUB):
        cps[c].wait()

        def sbody(s, _, _c=c):
            lo = jnp.maximum(jnp.max(jnp.where(lane == s, starts, 0)),
                             _c * SUB)
            hi = jnp.minimum(jnp.max(jnp.where(lane == s, ends, 0)),
                             (_c + 1) * SUB)

            @plsc.parallel_loop(lo, hi, unroll=8,
                                carry=tuple(zero16 for _ in range(NV)))
            def res(t, carry):
                wb = plsc.load_gather(lv, [jnp.zeros((16,), jnp.int32) + t])
                tl = t - _c * SUB
                return tuple(
                    carry[j] + wb * xb[_c % 2, tl, pl.ds(j * 16, 16)]
                    for j in range(NV))

            for j in range(NV):
                accv[s, pl.ds(j * 16, 16)] += res[j]
            return 0

        lax.fori_loop(s_first, s_last + 1, sbody, 0)
        if c + 2 < NSUB:
            cps[c + 2] = pltpu.async_copy(
                x_hbm.at[pl.ds(base + (c + 2) * SUB, SUB)],
                xb.at[c % 2], sems[c % 2])

    idxv[...] = lane

    @pl.when(sid == 0)
    def _seed():
        pltpu.sync_copy(accv, shacc)

    plsc.subcore_barrier()

    @pl.when(sid != 0)
    def _add():
        pltpu.sync_copy(accv, shacc.at[idxv], add=True)

    plsc.subcore_barrier()

    @pl.when(sid == 0)
    def _out():
        pltpu.sync_copy(shacc, accv)
        for s in range(B):
            db = plsc.load_gather(dv, [jnp.full((16,), s, jnp.int32)])
            rb = 1.0 / (db + 1e-16)
            for j in range(NV):
                accv[s, pl.ds(j * 16, 16)] *= rb
        pltpu.sync_copy(accv, out_hbm.at[cid])


_sc_reduce = pl.kernel(
    _sc_reduce_body,
    out_type=jax.ShapeDtypeStruct((NCORES, B, H), jnp.float32),
    mesh=plsc.VectorSubcoreMesh(core_axis_name="c", subcore_axis_name="s"),
    scratch_types=[
        pltpu.VMEM((2, SUB, H), jnp.float32),
        pltpu.VMEM((TOK,), jnp.float32),
        pltpu.VMEM((TOK,), jnp.int32),
        pltpu.VMEM((B,), jnp.float32),
        pltpu.VMEM((B,), jnp.float32),
        pltpu.VMEM((B, H), jnp.float32),
        pltpu.VMEM((B,), jnp.int32),
        pltpu.VMEM_SHARED((B, H), jnp.float32),
        pltpu.SemaphoreType.DMA,
        pltpu.SemaphoreType.DMA,
        pltpu.SemaphoreType.DMA,
    ],
    compiler_params=pltpu.CompilerParams(needs_layout_passes=False),
)


@jax.jit
def kernel(x, batch, last_ixs, W1, b1, W2, b2, qw, qb):
    segi = batch.astype(jnp.int32)
    seg3 = segi.reshape(NBLK, 1, BLK)
    b12 = (b1 + b2).reshape(1, H)
    qb2 = qb.reshape(1, 1)

    exw, s, hist = pl.pallas_call(
        _tc_body,
        grid=(NBLK,),
        in_specs=[
            pl.BlockSpec(memory_space=pltpu.SMEM),
            pl.BlockSpec(memory_space=pltpu.MemorySpace.HBM),
            pl.BlockSpec((BLK, H), lambda i: (i, 0)),
            pl.BlockSpec((1, 1, BLK), lambda i: (i, 0, 0)),
            pl.BlockSpec((H, H), lambda i: (0, 0)),
            pl.BlockSpec((1, H), lambda i: (0, 0)),
            pl.BlockSpec((H, H), lambda i: (0, 0)),
            pl.BlockSpec((H, 1), lambda i: (0, 0)),
            pl.BlockSpec((1, 1), lambda i: (0, 0)),
        ],
        out_specs=[
            pl.BlockSpec((1, 1, BLK), lambda i: (i, 0, 0)),
            pl.BlockSpec((B, 1), lambda i: (0, 0)),
            pl.BlockSpec((NCH, 1, B), lambda i: (i, 0, 0)),
        ],
        out_shape=[
            jax.ShapeDtypeStruct((NBLK, 1, BLK), jnp.float32),
            jax.ShapeDtypeStruct((B, 1), jnp.float32),
            jax.ShapeDtypeStruct((NTILES, 1, B), jnp.float32),
        ],
        scratch_shapes=[
            pltpu.VMEM((B, H), jnp.float32),
            pltpu.VMEM((B, 1), jnp.float32),
            pltpu.VMEM((B, H), jnp.float32),
            pltpu.SMEM((1,), jnp.float32),
            pltpu.SemaphoreType.DMA,
        ],
        compiler_params=pltpu.CompilerParams(
            dimension_semantics=("arbitrary",),
            fuse_transposed_lhs_in_matmul=True,
        ),
    )(last_ixs.astype(jnp.int32), x, x, seg3, W1, b12, W2, qw, qb2)

    partials = _sc_reduce(x, exw.reshape(N), segi, s.reshape(B),
                          hist.reshape(NTILES, B))
    return partials[0] + partials[1]


# speedup vs baseline: 2.2397x; 1.0117x over previous
import functools
import jax
import jax.numpy as jnp
from jax import lax
from jax.experimental import pallas as pl
from jax.experimental.pallas import tpu as pltpu
from jax.experimental.pallas import tpu_sc as plsc

B = 16
N = 32768
H = 128
BLK = 8192
NBLK = N // BLK
NCH = BLK // 1024
NCORES = 2
NTILES = 32
TOK = N // NTILES
SUB = 256
NSUB = TOK // SUB
NV = H // 16


def _tc_body(last_sm, x_any, x_ref, seg_ref, W1_ref, b12_ref, W2_ref, qw_ref,
             qb_ref, lg_ref, s_ref, hist_ref, u_sc, s_sc, vi_sc, usm,
             sem):
    i = pl.program_id(0)

    @pl.when(i == 0)
    def _init():
        cps = [
            pltpu.make_async_copy(x_any.at[pl.ds(last_sm[b], 1)],
                                  vi_sc.at[pl.ds(b, 1)], sem)
            for b in range(B)
        ]
        for cp in cps:
            cp.start()
        for cp in cps:
            cp.wait()
        u_sc[...] = jnp.dot(vi_sc[...], W1_ref[...],
                            preferred_element_type=jnp.float32) + b12_ref[...]
        usm[0] = jnp.sum(jnp.maximum(qw_ref[...], 0.0)) + qb_ref[0, 0]
        s_sc[...] = jnp.zeros((B, 1), jnp.float32)

    x = x_ref[...]
    seg_row = seg_ref[0]
    iota = lax.broadcasted_iota(jnp.int32, (B, BLK), 0)
    ohT = (seg_row == iota).astype(jnp.float32)

    z = jnp.dot(x, W2_ref[...], preferred_element_type=jnp.float32)
    z = z + lax.dot_general(ohT, u_sc[...], (((0,), (0,)), ((), ())),
                            preferred_element_type=jnp.float32)
    h = 0.5 * jnp.tanh(0.5 * z) + 0.5
    lgr = lax.dot_general(qw_ref[...], h, (((0,), (1,)), ((), ())),
                          preferred_element_type=jnp.float32) + qb_ref[...]

    ones_row = jnp.ones((1, 1024), jnp.float32)
    hist_ref[...] = jnp.concatenate(
        [lax.dot_general(ones_row, ohT[:, c * 1024:(c + 1) * 1024],
                         (((1,), (1,)), ((), ())),
                         preferred_element_type=jnp.float32)[None]
         for c in range(NCH)], axis=0)

    ex = jnp.exp(lgr - usm[0])
    lg_ref[...] = ex[None]
    s_sc[...] = s_sc[...] + jnp.sum(ohT * ex, axis=1, keepdims=True)

    @pl.when(i == NBLK - 1)
    def _fin():
        s_ref[...] = s_sc[...]


def _sc_reduce_body(x_hbm, lg_hbm, seg_hbm, d_hbm, hist_hbm, out_hbm,
                    xb, lv, segv, dv, histv, accv, idxv, shacc,
                    sem0, sem1, semh):
    cid = lax.axis_index("c")
    sid = lax.axis_index("s")
    wid = sid * NCORES + cid
    base = wid * TOK
    sems = [sem0, sem1]

    cps = [None] * NSUB
    for c in range(min(2, NSUB)):
        cps[c] = pltpu.async_copy(x_hbm.at[pl.ds(base + c * SUB, SUB)],
                                  xb.at[c % 2], sems[c % 2])

    hdr = [
        pltpu.async_copy(lg_hbm.at[pl.ds(base, TOK)], lv, semh),
        pltpu.async_copy(seg_hbm.at[pl.ds(base, TOK)], segv, semh),
        pltpu.async_copy(d_hbm, dv, semh),
        pltpu.async_copy(hist_hbm.at[wid], histv, semh),
    ]
    for cp in hdr:
        cp.wait()

    hist_i = histv[...].astype(jnp.int32)
    ends = jnp.cumsum(hist_i)
    starts = ends - hist_i
    lane = lax.iota(jnp.int32, 16)
    s_first = jnp.min(jnp.where(hist_i > 0, lane, B))
    s_last = jnp.max(jnp.where(hist_i > 0, lane, -1))

    zero16 = jnp.zeros((16,), jnp.float32)
    for r in range(B):
        for j in range(NV):
            accv[r, pl.ds(j * 16, 16)] = zero16

    for c in range(NSUB):
        cps[c].wait()

        def sbody(s, _, _c=c):
            lo = jnp.maximum(jnp.max(jnp.where(lane == s, starts, 0)),
                             _c * SUB)
            hi = jnp.minimum(jnp.max(jnp.where(lane == s, ends, 0)),
                             (_c + 1) * SUB)

            @plsc.parallel_loop(lo, hi, unroll=8,
                                carry=tuple(zero16 for _ in range(NV)))
            def res(t, carry):
                wb = plsc.load_gather(lv, [jnp.zeros((16,), jnp.int32) + t])
                tl = t - _c * SUB
                return tuple(
                    carry[j] + wb * xb[_c % 2, tl, pl.ds(j * 16, 16)]
                    for j in range(NV))

            for j in range(NV):
                accv[s, pl.ds(j * 16, 16)] += res[j]
            return 0

        lax.fori_loop(s_first, s_last + 1, sbody, 0)
        if c + 2 < NSUB:
            cps[c + 2] = pltpu.async_copy(
                x_hbm.at[pl.ds(base + (c + 2) * SUB, SUB)],
                xb.at[c % 2], sems[c % 2])

    def dbody(s, _):
        db = plsc.load_gather(dv, [jnp.zeros((16,), jnp.int32) + s])
        for j in range(NV):
            accv[s, pl.ds(j * 16, 16)] /= db + 1e-16
        return 0

    lax.fori_loop(s_first, s_last + 1, dbody, 0)

    idxv[...] = lane

    @pl.when(sid == 0)
    def _seed():
        pltpu.sync_copy(accv, shacc)

    plsc.subcore_barrier()

    @pl.when(sid != 0)
    def _add():
        pltpu.sync_copy(accv, shacc.at[idxv], add=True)

    plsc.subcore_barrier()

    @pl.when(sid == 0)
    def _out():
        pltpu.sync_copy(shacc, out_hbm.at[cid])


_sc_reduce = pl.kernel(
    _sc_reduce_body,
    out_type=jax.ShapeDtypeStruct((NCORES, B, H), jnp.float32),
    mesh=plsc.VectorSubcoreMesh(core_axis_name="c", subcore_axis_name="s"),
    scratch_types=[
        pltpu.VMEM((2, SUB, H), jnp.float32),
        pltpu.VMEM((TOK,), jnp.float32),
        pltpu.VMEM((TOK,), jnp.int32),
        pltpu.VMEM((B,), jnp.float32),
        pltpu.VMEM((B,), jnp.float32),
        pltpu.VMEM((B, H), jnp.float32),
        pltpu.VMEM((B,), jnp.int32),
        pltpu.VMEM_SHARED((B, H), jnp.float32),
        pltpu.SemaphoreType.DMA,
        pltpu.SemaphoreType.DMA,
        pltpu.SemaphoreType.DMA,
    ],
    compiler_params=pltpu.CompilerParams(needs_layout_passes=False),
)


@jax.jit
def kernel(x, batch, last_ixs, W1, b1, W2, b2, qw, qb):
    segi = batch.astype(jnp.int32)
    seg3 = segi.reshape(NBLK, 1, BLK)
    b12 = (b1 + b2).reshape(1, H)
    qb2 = qb.reshape(1, 1)

    exw, s, hist = pl.pallas_call(
        _tc_body,
        grid=(NBLK,),
        in_specs=[
            pl.BlockSpec(memory_space=pltpu.SMEM),
            pl.BlockSpec(memory_space=pltpu.MemorySpace.HBM),
            pl.BlockSpec((BLK, H), lambda i: (i, 0)),
            pl.BlockSpec((1, 1, BLK), lambda i: (i, 0, 0)),
            pl.BlockSpec((H, H), lambda i: (0, 0)),
            pl.BlockSpec((1, H), lambda i: (0, 0)),
            pl.BlockSpec((H, H), lambda i: (0, 0)),
            pl.BlockSpec((H, 1), lambda i: (0, 0)),
            pl.BlockSpec((1, 1), lambda i: (0, 0)),
        ],
        out_specs=[
            pl.BlockSpec((1, 1, BLK), lambda i: (i, 0, 0)),
            pl.BlockSpec((B, 1), lambda i: (0, 0)),
            pl.BlockSpec((NCH, 1, B), lambda i: (i, 0, 0)),
        ],
        out_shape=[
            jax.ShapeDtypeStruct((NBLK, 1, BLK), jnp.float32),
            jax.ShapeDtypeStruct((B, 1), jnp.float32),
            jax.ShapeDtypeStruct((NTILES, 1, B), jnp.float32),
        ],
        scratch_shapes=[
            pltpu.VMEM((B, H), jnp.float32),
            pltpu.VMEM((B, 1), jnp.float32),
            pltpu.VMEM((B, H), jnp.float32),
            pltpu.SMEM((1,), jnp.float32),
            pltpu.SemaphoreType.DMA,
        ],
        compiler_params=pltpu.CompilerParams(
            dimension_semantics=("arbitrary",),
            fuse_transposed_lhs_in_matmul=True,
        ),
    )(last_ixs.astype(jnp.int32), x, x, seg3, W1, b12, W2, qw, qb2)

    partials = _sc_reduce(x, exw.reshape(N), segi, s.reshape(B),
                          hist.reshape(NTILES, B))
    return partials[0] + partials[1]
